# Initial kernel scaffold; baseline (speedup 1.0000x reference)
import jax
import jax.numpy as jnp
from jax.experimental import pallas as pl


def kernel(x, edge_index, W1, b1, W2, b2):
    raise NotImplementedError("write your pallas kernel here")


# speedup vs baseline: 17.0002x; 17.0002x over previous
import functools

import jax
import jax.numpy as jnp
from jax import lax
from jax.experimental import pallas as pl
from jax.experimental.pallas import tpu as pltpu
from jax.experimental.pallas import tpu_sc as plsc

N = 10000
E = 320000
D = 128
H = 128

NC = 2
NS = 16
NW = NC * NS
NPAD = 10240
RPT = NPAD // NS
EPW = E // NW
CH = 128
NCHUNK = EPW // CH
TAIL = EPW - NCHUNK * CH

_f32 = jnp.float32
_mesh = plsc.VectorSubcoreMesh(core_axis_name="c", subcore_axis_name="s")


def _zero_vmem_1d(ref, n):
    def body(i, carry):
        ref[pl.ds(i * 16, 16)] = jnp.zeros((16,), _f32)
        return carry
    lax.fori_loop(0, n // 16, body, 0)


def _fill_vmem_1d(ref, n, val):
    def body(i, carry):
        ref[pl.ds(i * 16, 16)] = jnp.full((16,), val, _f32)
        return carry
    lax.fori_loop(0, n // 16, body, 0)


def _zero_vmem_2d(ref, rows):
    def body(i, carry):
        for j in range(D // 16):
            ref[i, pl.ds(j * 16, 16)] = jnp.zeros((16,), _f32)
        return carry
    lax.fori_loop(0, rows, body, 0)


def _deg_body(dst_hbm, out_hbm, idx_v, idx_t, ones_v, ones_t, zb, deg_sh):
    c = lax.axis_index("c")
    s = lax.axis_index("s")
    w = s * NC + c

    _zero_vmem_1d(zb, RPT)
    pltpu.sync_copy(zb, deg_sh.at[pl.ds(s * RPT, RPT)])
    _fill_vmem_1d(ones_v, CH, 1.0)
    _fill_vmem_1d(ones_t, TAIL, 1.0)
    plsc.subcore_barrier()

    base = w * EPW

    def chunk(j, carry):
        pltpu.sync_copy(dst_hbm.at[pl.ds(base + j * CH, CH)], idx_v)
        pltpu.sync_copy(ones_v, deg_sh.at[idx_v], add=True)
        return carry

    lax.fori_loop(0, NCHUNK, chunk, 0)
    pltpu.sync_copy(dst_hbm.at[pl.ds(base + NCHUNK * CH, TAIL)], idx_t)
    pltpu.sync_copy(ones_t, deg_sh.at[idx_t], add=True)

    plsc.subcore_barrier()
    pltpu.sync_copy(deg_sh.at[pl.ds(s * RPT, RPT)],
                    out_hbm.at[c, pl.ds(s * RPT, RPT)])


_deg_kernel = functools.partial(
    pl.kernel,
    out_type=jax.ShapeDtypeStruct((NC, NPAD), _f32),
    mesh=_mesh,
    scratch_types=[
        pltpu.VMEM((CH,), jnp.int32),
        pltpu.VMEM((TAIL,), jnp.int32),
        pltpu.VMEM((CH,), _f32),
        pltpu.VMEM((TAIL,), _f32),
        pltpu.VMEM((RPT,), _f32),
        pltpu.VMEM_SHARED((NPAD,), _f32),
    ],
)(_deg_body)


def _agg_body(y_hbm, src_hbm, dst_hbm, out_hbm,
              sidx, didx, sidx_t, didx_t, rows, rows_t, zb, acc_sh, sem):
    c = lax.axis_index("c")
    s = lax.axis_index("s")
    w = s * NC + c

    _zero_vmem_2d(zb, 64)

    def zcp(t, carry):
        pltpu.sync_copy(zb, acc_sh.at[pl.ds(s * RPT + t * 64, 64)])
        return carry

    lax.fori_loop(0, RPT // 64, zcp, 0)
    plsc.subcore_barrier()

    base = w * EPW

    def chunk(j, carry):
        pltpu.sync_copy(src_hbm.at[pl.ds(base + j * CH, CH)], sidx)
        pltpu.sync_copy(dst_hbm.at[pl.ds(base + j * CH, CH)], didx)
        pltpu.async_copy(y_hbm.at[sidx], rows, sem).wait()
        pltpu.sync_copy(rows, acc_sh.at[didx], add=True)
        return carry

    lax.fori_loop(0, NCHUNK, chunk, 0)
    pltpu.sync_copy(src_hbm.at[pl.ds(base + NCHUNK * CH, TAIL)], sidx_t)
    pltpu.sync_copy(dst_hbm.at[pl.ds(base + NCHUNK * CH, TAIL)], didx_t)
    pltpu.async_copy(y_hbm.at[sidx_t], rows_t, sem).wait()
    pltpu.sync_copy(rows_t, acc_sh.at[didx_t], add=True)

    plsc.subcore_barrier()
    pltpu.sync_copy(acc_sh.at[pl.ds(s * RPT, RPT)],
                    out_hbm.at[c, pl.ds(s * RPT, RPT)])


_agg_kernel = functools.partial(
    pl.kernel,
    out_type=jax.ShapeDtypeStruct((NC, NPAD, D), _f32),
    mesh=_mesh,
    scratch_types=[
        pltpu.VMEM((CH,), jnp.int32),
        pltpu.VMEM((CH,), jnp.int32),
        pltpu.VMEM((TAIL,), jnp.int32),
        pltpu.VMEM((TAIL,), jnp.int32),
        pltpu.VMEM((CH, D), _f32),
        pltpu.VMEM((TAIL, D), _f32),
        pltpu.VMEM((64, D), _f32),
        pltpu.VMEM_SHARED((NPAD, D), _f32),
        pltpu.SemaphoreType.DMA,
    ],
)(_agg_body)


def _d_of(dp):
    return lax.rsqrt(jnp.maximum(dp[0] + dp[1] + 1.0, 1.0))


def _y1_body(x_ref, w1_ref, dp_ref, o_ref):
    d = _d_of(dp_ref[...])
    o_ref[...] = d * jnp.dot(x_ref[...], w1_ref[...],
                             preferred_element_type=_f32)


def _mid_body(dp_ref, y1_ref, p_ref, b1_ref, w2_ref, o_ref):
    d = _d_of(dp_ref[...])
    p = p_ref[...]
    h = jnp.maximum(d * (p[0] + p[1] + y1_ref[...]) + b1_ref[...], 0.0)
    o_ref[...] = d * jnp.dot(h, w2_ref[...], preferred_element_type=_f32)


def _out_body(dp_ref, y2_ref, p_ref, b2_ref, o_ref):
    d = _d_of(dp_ref[...])
    p = p_ref[...]
    o_ref[...] = d * (p[0] + p[1] + y2_ref[...]) + b2_ref[...]


_y1_call = pl.pallas_call(
    _y1_body, out_shape=jax.ShapeDtypeStruct((NPAD, H), _f32))
_mid_call = pl.pallas_call(
    _mid_body, out_shape=jax.ShapeDtypeStruct((NPAD, H), _f32))
_out_call = pl.pallas_call(
    _out_body, out_shape=jax.ShapeDtypeStruct((NPAD, H), _f32))


def kernel(x, edge_index, W1, b1, W2, b2):
    src = edge_index[0]
    dst = edge_index[1]
    x_pad = jnp.pad(x, ((0, NPAD - N), (0, 0)))

    degp = _deg_kernel(dst)
    dp = degp.reshape(NC, NPAD, 1)

    y1 = _y1_call(x_pad, W1, dp)
    p1 = _agg_kernel(y1, src, dst)
    y2 = _mid_call(dp, y1, p1, b1, W2)
    p2 = _agg_kernel(y2, src, dst)
    out = _out_call(dp, y2, p2, b2)
    return out[:N]


# speedup vs baseline: 24.7292x; 1.4546x over previous
import functools

import jax
import jax.numpy as jnp
from jax import lax
from jax.experimental import pallas as pl
from jax.experimental.pallas import tpu as pltpu
from jax.experimental.pallas import tpu_sc as plsc

N = 10000
E = 320000
D = 128
H = 128

NC = 2
NS = 16
NW = NC * NS
NPAD = 10240
RPT = NPAD // NS
EPW = E // NW
CH = 128
NCHUNK = EPW // CH
TAIL = EPW - NCHUNK * CH

_f32 = jnp.float32
_mesh = plsc.VectorSubcoreMesh(core_axis_name="c", subcore_axis_name="s")


def _zero_vmem_1d(ref, n):
    def body(i, carry):
        ref[pl.ds(i * 16, 16)] = jnp.zeros((16,), _f32)
        return carry
    lax.fori_loop(0, n // 16, body, 0)


def _fill_vmem_1d(ref, n, val):
    def body(i, carry):
        ref[pl.ds(i * 16, 16)] = jnp.full((16,), val, _f32)
        return carry
    lax.fori_loop(0, n // 16, body, 0)


def _zero_vmem_2d(ref, rows):
    def body(i, carry):
        for j in range(D // 16):
            ref[i, pl.ds(j * 16, 16)] = jnp.zeros((16,), _f32)
        return carry
    lax.fori_loop(0, rows, body, 0)


def _deg_body(dst_hbm, out_hbm, idx_v, idx_t, ones_v, ones_t, zb, deg_sh):
    c = lax.axis_index("c")
    s = lax.axis_index("s")
    w = s * NC + c

    _zero_vmem_1d(zb, RPT)
    pltpu.sync_copy(zb, deg_sh.at[pl.ds(s * RPT, RPT)])
    _fill_vmem_1d(ones_v, CH, 1.0)
    _fill_vmem_1d(ones_t, TAIL, 1.0)
    plsc.subcore_barrier()

    base = w * EPW

    def chunk(j, carry):
        pltpu.sync_copy(dst_hbm.at[pl.ds(base + j * CH, CH)], idx_v)
        pltpu.sync_copy(ones_v, deg_sh.at[idx_v], add=True)
        return carry

    lax.fori_loop(0, NCHUNK, chunk, 0)
    pltpu.sync_copy(dst_hbm.at[pl.ds(base + NCHUNK * CH, TAIL)], idx_t)
    pltpu.sync_copy(ones_t, deg_sh.at[idx_t], add=True)

    plsc.subcore_barrier()
    pltpu.sync_copy(deg_sh.at[pl.ds(s * RPT, RPT)],
                    out_hbm.at[c, pl.ds(s * RPT, RPT)])


_deg_kernel = functools.partial(
    pl.kernel,
    out_type=jax.ShapeDtypeStruct((NC, NPAD), _f32),
    mesh=_mesh,
    scratch_types=[
        pltpu.VMEM((CH,), jnp.int32),
        pltpu.VMEM((TAIL,), jnp.int32),
        pltpu.VMEM((CH,), _f32),
        pltpu.VMEM((TAIL,), _f32),
        pltpu.VMEM((RPT,), _f32),
        pltpu.VMEM_SHARED((NPAD,), _f32),
    ],
)(_deg_body)


NPAIR = NCHUNK // 2


def _agg_body(y_hbm, src_hbm, dst_hbm, out_hbm,
              sidx0, sidx1, didx0, didx1, sidx_t, didx_t,
              rows0, rows1, rows_t, zb, acc_sh,
              gsem0, gsem1, ssem0, ssem1, sem_t):
    c = lax.axis_index("c")
    s = lax.axis_index("s")
    w = s * NC + c

    _zero_vmem_2d(zb, 64)

    def zcp(t, carry):
        pltpu.sync_copy(zb, acc_sh.at[pl.ds(s * RPT + t * 64, 64)])
        return carry

    lax.fori_loop(0, RPT // 64, zcp, 0)
    plsc.subcore_barrier()

    base = w * EPW

    def ld_idx(j, si, di):
        pltpu.sync_copy(src_hbm.at[pl.ds(base + j * CH, CH)], si)
        pltpu.sync_copy(dst_hbm.at[pl.ds(base + j * CH, CH)], di)

    ld_idx(0, sidx0, didx0)
    pltpu.async_copy(y_hbm.at[sidx0], rows0, gsem0)

    def pair(g, carry):
        j0 = 2 * g

        @pl.when(g > 0)
        def _():
            pltpu.make_async_copy(rows1, acc_sh.at[didx1], ssem1).wait()

        ld_idx(j0 + 1, sidx1, didx1)
        pltpu.async_copy(y_hbm.at[sidx1], rows1, gsem1)
        pltpu.make_async_copy(y_hbm.at[sidx0], rows0, gsem0).wait()
        pltpu.async_copy(rows0, acc_sh.at[didx0], ssem0, add=True)
        pltpu.make_async_copy(rows0, acc_sh.at[didx0], ssem0).wait()

        @pl.when(g < NPAIR - 1)
        def _():
            ld_idx(j0 + 2, sidx0, didx0)
            pltpu.async_copy(y_hbm.at[sidx0], rows0, gsem0)

        pltpu.make_async_copy(y_hbm.at[sidx1], rows1, gsem1).wait()
        pltpu.async_copy(rows1, acc_sh.at[didx1], ssem1, add=True)
        return carry

    lax.fori_loop(0, NPAIR, pair, 0)
    pltpu.make_async_copy(rows1, acc_sh.at[didx1], ssem1).wait()

    pltpu.sync_copy(src_hbm.at[pl.ds(base + NCHUNK * CH, TAIL)], sidx_t)
    pltpu.sync_copy(dst_hbm.at[pl.ds(base + NCHUNK * CH, TAIL)], didx_t)
    pltpu.async_copy(y_hbm.at[sidx_t], rows_t, sem_t).wait()
    pltpu.sync_copy(rows_t, acc_sh.at[didx_t], add=True)

    plsc.subcore_barrier()
    pltpu.sync_copy(acc_sh.at[pl.ds(s * RPT, RPT)],
                    out_hbm.at[c, pl.ds(s * RPT, RPT)])


_agg_kernel = functools.partial(
    pl.kernel,
    out_type=jax.ShapeDtypeStruct((NC, NPAD, D), _f32),
    mesh=_mesh,
    scratch_types=[
        pltpu.VMEM((CH,), jnp.int32),
        pltpu.VMEM((CH,), jnp.int32),
        pltpu.VMEM((CH,), jnp.int32),
        pltpu.VMEM((CH,), jnp.int32),
        pltpu.VMEM((TAIL,), jnp.int32),
        pltpu.VMEM((TAIL,), jnp.int32),
        pltpu.VMEM((CH, D), _f32),
        pltpu.VMEM((CH, D), _f32),
        pltpu.VMEM((TAIL, D), _f32),
        pltpu.VMEM((64, D), _f32),
        pltpu.VMEM_SHARED((NPAD, D), _f32),
        pltpu.SemaphoreType.DMA,
        pltpu.SemaphoreType.DMA,
        pltpu.SemaphoreType.DMA,
        pltpu.SemaphoreType.DMA,
        pltpu.SemaphoreType.DMA,
    ],
)(_agg_body)


def _d_of(dp):
    return lax.rsqrt(jnp.maximum(dp[0] + dp[1] + 1.0, 1.0))


def _y1_body(x_ref, w1_ref, dp_ref, o_ref):
    d = _d_of(dp_ref[...])
    o_ref[...] = d * jnp.dot(x_ref[...], w1_ref[...],
                             preferred_element_type=_f32)


def _mid_body(dp_ref, y1_ref, p_ref, b1_ref, w2_ref, o_ref):
    d = _d_of(dp_ref[...])
    p = p_ref[...]
    h = jnp.maximum(d * (p[0] + p[1] + y1_ref[...]) + b1_ref[...], 0.0)
    o_ref[...] = d * jnp.dot(h, w2_ref[...], preferred_element_type=_f32)


def _out_body(dp_ref, y2_ref, p_ref, b2_ref, o_ref):
    d = _d_of(dp_ref[...])
    p = p_ref[...]
    o_ref[...] = d * (p[0] + p[1] + y2_ref[...]) + b2_ref[...]


_y1_call = pl.pallas_call(
    _y1_body, out_shape=jax.ShapeDtypeStruct((NPAD, H), _f32))
_mid_call = pl.pallas_call(
    _mid_body, out_shape=jax.ShapeDtypeStruct((NPAD, H), _f32))
_out_call = pl.pallas_call(
    _out_body, out_shape=jax.ShapeDtypeStruct((NPAD, H), _f32))


def kernel(x, edge_index, W1, b1, W2, b2):
    src = edge_index[0]
    dst = edge_index[1]
    x_pad = jnp.pad(x, ((0, NPAD - N), (0, 0)))

    degp = _deg_kernel(dst)
    dp = degp.reshape(NC, NPAD, 1)

    y1 = _y1_call(x_pad, W1, dp)
    p1 = _agg_kernel(y1, src, dst)
    y2 = _mid_call(dp, y1, p1, b1, W2)
    p2 = _agg_kernel(y2, src, dst)
    out = _out_call(dp, y2, p2, b2)
    return out[:N]


# speedup vs baseline: 29.4330x; 1.1902x over previous
import functools

import jax
import jax.numpy as jnp
from jax import lax
from jax.experimental import pallas as pl
from jax.experimental.pallas import tpu as pltpu
from jax.experimental.pallas import tpu_sc as plsc

N = 10000
E = 320000
D = 128
H = 128

NC = 2
NS = 16
NW = NC * NS
NPAD = 10240
RPT = NPAD // NS
CH = 64
NCH = 160
EPW = NCH * CH
E2 = EPW * NW
NPAIR = NCH // 2

_f32 = jnp.float32
_mesh = plsc.VectorSubcoreMesh(core_axis_name="c", subcore_axis_name="s")


def _fill_vmem_1d(ref, n, val):
    def body(i, carry):
        ref[pl.ds(i * 16, 16)] = jnp.full((16,), val, _f32)
        return carry
    lax.fori_loop(0, n // 16, body, 0)


def _zero_vmem_2d(ref, rows):
    def body(i, carry):
        for j in range(D // 16):
            ref[i, pl.ds(j * 16, 16)] = jnp.zeros((16,), _f32)
        return carry
    lax.fori_loop(0, rows, body, 0)


def _deg_body(dst2d_hbm, out_hbm, didx_all, ones_v, zb, deg_sh, dsem):
    c = lax.axis_index("c")
    s = lax.axis_index("s")
    w = s * NC + c

    pltpu.sync_copy(dst2d_hbm.at[pl.ds(w * NCH, NCH)], didx_all)
    _fill_vmem_1d(ones_v, CH, 1.0)
    _fill_vmem_1d(zb, RPT, 0.0)
    pltpu.sync_copy(zb, deg_sh.at[pl.ds(s * RPT, RPT)])
    plsc.subcore_barrier()

    def chunk(j, carry):
        pltpu.async_copy(ones_v, deg_sh.at[didx_all.at[j]], dsem, add=True)

        @pl.when(j >= 8)
        def _():
            pltpu.make_async_copy(ones_v, deg_sh.at[didx_all.at[j - 8]],
                                  dsem).wait()
        return carry

    lax.fori_loop(0, NCH, chunk, 0)
    for k in range(8):
        pltpu.make_async_copy(ones_v, deg_sh.at[didx_all.at[NCH - 8 + k]],
                              dsem).wait()

    plsc.subcore_barrier()
    pltpu.sync_copy(deg_sh.at[pl.ds(s * RPT, RPT)],
                    out_hbm.at[c, pl.ds(s * RPT, RPT)])


_deg_kernel = functools.partial(
    pl.kernel,
    out_type=jax.ShapeDtypeStruct((NC, NPAD), _f32),
    mesh=_mesh,
    scratch_types=[
        pltpu.VMEM((NCH, CH), jnp.int32),
        pltpu.VMEM((CH,), _f32),
        pltpu.VMEM((RPT,), _f32),
        pltpu.VMEM_SHARED((NPAD,), _f32),
        pltpu.SemaphoreType.DMA,
    ],
)(_deg_body)


def _agg_body(y_hbm, src_hbm, dst2d_hbm, out_hbm,
              sidx_all, didx_all, rows0, rows1, zb, acc_sh,
              gsem0, gsem1, ssem0, ssem1):
    c = lax.axis_index("c")
    s = lax.axis_index("s")
    w = s * NC + c

    pltpu.sync_copy(src_hbm.at[pl.ds(w * EPW, EPW)], sidx_all)
    pltpu.sync_copy(dst2d_hbm.at[pl.ds(w * NCH, NCH)], didx_all)

    _zero_vmem_2d(zb, 16)

    def zcp(t, carry):
        pltpu.sync_copy(zb, acc_sh.at[pl.ds(s * RPT + t * 16, 16)])
        return carry

    lax.fori_loop(0, RPT // 16, zcp, 0)
    plsc.subcore_barrier()

    def sidx(j):
        return sidx_all.at[pl.ds(j * CH, CH)]

    pltpu.async_copy(y_hbm.at[sidx(0)], rows0, gsem0)

    def pair(g, carry):
        j0 = 2 * g

        @pl.when(g > 0)
        def _():
            pltpu.make_async_copy(rows1, acc_sh.at[didx_all.at[j0 - 1]],
                                  ssem1).wait()

        pltpu.async_copy(y_hbm.at[sidx(j0 + 1)], rows1, gsem1)
        pltpu.make_async_copy(y_hbm.at[sidx(j0)], rows0, gsem0).wait()
        pltpu.async_copy(rows0, acc_sh.at[didx_all.at[j0]], ssem0, add=True)
        pltpu.make_async_copy(rows0, acc_sh.at[didx_all.at[j0]], ssem0).wait()

        @pl.when(g < NPAIR - 1)
        def _():
            pltpu.async_copy(y_hbm.at[sidx(j0 + 2)], rows0, gsem0)

        pltpu.make_async_copy(y_hbm.at[sidx(j0 + 1)], rows1, gsem1).wait()
        pltpu.async_copy(rows1, acc_sh.at[didx_all.at[j0 + 1]], ssem1,
                         add=True)
        return carry

    lax.fori_loop(0, NPAIR, pair, 0)
    pltpu.make_async_copy(rows1, acc_sh.at[didx_all.at[NCH - 1]],
                          ssem1).wait()

    plsc.subcore_barrier()
    pltpu.sync_copy(acc_sh.at[pl.ds(s * RPT, RPT)],
                    out_hbm.at[c, pl.ds(s * RPT, RPT)])


_agg_kernel = functools.partial(
    pl.kernel,
    out_type=jax.ShapeDtypeStruct((NC, NPAD, D), _f32),
    mesh=_mesh,
    scratch_types=[
        pltpu.VMEM((EPW,), jnp.int32),
        pltpu.VMEM((NCH, CH), jnp.int32),
        pltpu.VMEM((CH, D), _f32),
        pltpu.VMEM((CH, D), _f32),
        pltpu.VMEM((16, D), _f32),
        pltpu.VMEM_SHARED((NPAD, D), _f32),
        pltpu.SemaphoreType.DMA,
        pltpu.SemaphoreType.DMA,
        pltpu.SemaphoreType.DMA,
        pltpu.SemaphoreType.DMA,
    ],
)(_agg_body)


def _d_of(dp):
    return lax.rsqrt(jnp.maximum(dp[0] + dp[1] + 1.0, 1.0))


def _y1_body(x_ref, w1_ref, dp_ref, o_ref):
    d = _d_of(dp_ref[...])
    o_ref[...] = d * jnp.dot(x_ref[...], w1_ref[...],
                             preferred_element_type=_f32)


def _mid_body(dp_ref, y1_ref, p_ref, b1_ref, w2_ref, o_ref):
    d = _d_of(dp_ref[...])
    p = p_ref[...]
    h = jnp.maximum(d * (p[0] + p[1] + y1_ref[...]) + b1_ref[...], 0.0)
    o_ref[...] = d * jnp.dot(h, w2_ref[...], preferred_element_type=_f32)


def _out_body(dp_ref, y2_ref, p_ref, b2_ref, o_ref):
    d = _d_of(dp_ref[...])
    p = p_ref[...]
    o_ref[...] = d * (p[0] + p[1] + y2_ref[...]) + b2_ref[...]


_y1_call = pl.pallas_call(
    _y1_body, out_shape=jax.ShapeDtypeStruct((NPAD, H), _f32))
_mid_call = pl.pallas_call(
    _mid_body, out_shape=jax.ShapeDtypeStruct((NPAD, H), _f32))
_out_call = pl.pallas_call(
    _out_body, out_shape=jax.ShapeDtypeStruct((NPAD, H), _f32))


def kernel(x, edge_index, W1, b1, W2, b2):
    src = edge_index[0]
    dst = edge_index[1]
    x_pad = jnp.pad(x, ((0, NPAD - N), (0, 0)))

    pad = E2 - E
    ar = jnp.arange(pad, dtype=jnp.int32)
    src_p = jnp.concatenate([src, (ar * 37) % N])
    dst_p = jnp.concatenate([dst, N + (ar % (NPAD - N))])
    dst2d = dst_p.reshape(E2 // CH, CH)

    degp = _deg_kernel(dst2d)
    dp = degp.reshape(NC, NPAD, 1)

    y1 = _y1_call(x_pad, W1, dp)
    p1 = _agg_kernel(y1, src_p, dst2d)
    y2 = _mid_call(dp, y1, p1, b1, W2)
    p2 = _agg_kernel(y2, src_p, dst2d)
    out = _out_call(dp, y2, p2, b2)
    return out[:N]


# speedup vs baseline: 30.9040x; 1.0500x over previous
import functools

import jax
import jax.numpy as jnp
from jax import lax
from jax.experimental import pallas as pl
from jax.experimental.pallas import tpu as pltpu
from jax.experimental.pallas import tpu_sc as plsc

N = 10000
E = 320000
D = 128
H = 128

NC = 2
NS = 16
NW = NC * NS
NPAD = 10240
RPT = NPAD // NS
CH = 64
NCH = 160
HCH = NCH // 2
EPW = NCH * CH
E2 = EPW * NW

_f32 = jnp.float32
_mesh = plsc.VectorSubcoreMesh(core_axis_name="c", subcore_axis_name="s")


def _fill_vmem_1d(ref, n, val):
    def body(i, carry):
        ref[pl.ds(i * 16, 16)] = jnp.full((16,), val, _f32)
        return carry
    lax.fori_loop(0, n // 16, body, 0)


def _zero_vmem_2d(ref, rows):
    def body(i, carry):
        for j in range(D // 16):
            ref[i, pl.ds(j * 16, 16)] = jnp.zeros((16,), _f32)
        return carry
    lax.fori_loop(0, rows, body, 0)


def _deg_body(dst2d_hbm, out_hbm, didx_all, ones_v, zb, deg_sh, dsem):
    c = lax.axis_index("c")
    s = lax.axis_index("s")
    w = s * NC + c

    pltpu.sync_copy(dst2d_hbm.at[pl.ds(w * NCH, NCH)], didx_all)
    _fill_vmem_1d(ones_v, CH, 1.0)
    _fill_vmem_1d(zb, RPT, 0.0)
    pltpu.sync_copy(zb, deg_sh.at[pl.ds(s * RPT, RPT)])
    plsc.subcore_barrier()

    def chunk(j, carry):
        pltpu.async_copy(ones_v, deg_sh.at[didx_all.at[j]], dsem, add=True)

        @pl.when(j >= 8)
        def _():
            pltpu.make_async_copy(ones_v, deg_sh.at[didx_all.at[j - 8]],
                                  dsem).wait()
        return carry

    lax.fori_loop(0, NCH, chunk, 0)
    for k in range(8):
        pltpu.make_async_copy(ones_v, deg_sh.at[didx_all.at[NCH - 8 + k]],
                              dsem).wait()

    plsc.subcore_barrier()
    pltpu.sync_copy(deg_sh.at[pl.ds(s * RPT, RPT)],
                    out_hbm.at[c, pl.ds(s * RPT, RPT)])


_deg_kernel = functools.partial(
    pl.kernel,
    out_type=jax.ShapeDtypeStruct((NC, NPAD), _f32),
    mesh=_mesh,
    scratch_types=[
        pltpu.VMEM((NCH, CH), jnp.int32),
        pltpu.VMEM((CH,), _f32),
        pltpu.VMEM((RPT,), _f32),
        pltpu.VMEM_SHARED((NPAD,), _f32),
        pltpu.SemaphoreType.DMA,
    ],
)(_deg_body)


def _agg_body(y_hbm, src_hbm, dst2d_hbm, out_hbm,
              sidx_h, didx_all, rows0, rows1, rows2, rows3, acc_sh,
              gsem0, gsem1, gsem2, gsem3, ssem0, ssem1, ssem2, ssem3):
    c = lax.axis_index("c")
    s = lax.axis_index("s")
    w = s * NC + c

    rows = (rows0, rows1, rows2, rows3)
    gsem = (gsem0, gsem1, gsem2, gsem3)
    ssem = (ssem0, ssem1, ssem2, ssem3)

    _zero_vmem_2d(rows0, 64)

    def zcp(t, carry):
        pltpu.sync_copy(rows0, acc_sh.at[pl.ds(s * RPT + t * 64, 64)])
        return carry

    lax.fori_loop(0, RPT // 64, zcp, 0)
    plsc.subcore_barrier()

    def gsl(l):
        return sidx_h.at[pl.ds(l * CH, CH)]

    for h in range(2):
        jbase = h * HCH
        pltpu.sync_copy(src_hbm.at[pl.ds(w * EPW + jbase * CH, HCH * CH)],
                        sidx_h)
        pltpu.sync_copy(dst2d_hbm.at[pl.ds(w * NCH + jbase, HCH)], didx_all)
        pltpu.async_copy(y_hbm.at[gsl(0)], rows0, gsem0)
        pltpu.async_copy(y_hbm.at[gsl(1)], rows1, gsem1)

        def quad(t, carry):
            for u in range(4):
                l = 4 * t + u
                b, b2 = u, (u + 2) % 4
                pltpu.make_async_copy(y_hbm.at[gsl(l)], rows[b],
                                      gsem[b]).wait()
                pltpu.async_copy(rows[b], acc_sh.at[didx_all.at[l]],
                                 ssem[b], add=True)

                @pl.when(l >= 2)
                def _():
                    pltpu.make_async_copy(rows[b2],
                                          acc_sh.at[didx_all.at[l - 2]],
                                          ssem[b2]).wait()

                @pl.when(l + 2 < HCH)
                def _():
                    pltpu.async_copy(y_hbm.at[gsl(l + 2)], rows[b2], gsem[b2])
            return carry

        lax.fori_loop(0, HCH // 4, quad, 0)
        pltpu.make_async_copy(rows2, acc_sh.at[didx_all.at[HCH - 2]],
                              ssem2).wait()
        pltpu.make_async_copy(rows3, acc_sh.at[didx_all.at[HCH - 1]],
                              ssem3).wait()

    plsc.subcore_barrier()
    pltpu.sync_copy(acc_sh.at[pl.ds(s * RPT, RPT)],
                    out_hbm.at[c, pl.ds(s * RPT, RPT)])


_agg_kernel = functools.partial(
    pl.kernel,
    out_type=jax.ShapeDtypeStruct((NC, NPAD, D), _f32),
    mesh=_mesh,
    scratch_types=[
        pltpu.VMEM((HCH * CH,), jnp.int32),
        pltpu.VMEM((HCH, CH), jnp.int32),
        pltpu.VMEM((CH, D), _f32),
        pltpu.VMEM((CH, D), _f32),
        pltpu.VMEM((CH, D), _f32),
        pltpu.VMEM((CH, D), _f32),
        pltpu.VMEM_SHARED((NPAD, D), _f32),
        pltpu.SemaphoreType.DMA,
        pltpu.SemaphoreType.DMA,
        pltpu.SemaphoreType.DMA,
        pltpu.SemaphoreType.DMA,
        pltpu.SemaphoreType.DMA,
        pltpu.SemaphoreType.DMA,
        pltpu.SemaphoreType.DMA,
        pltpu.SemaphoreType.DMA,
    ],
)(_agg_body)


def _d_of(dp):
    return lax.rsqrt(jnp.maximum(dp[0] + dp[1] + 1.0, 1.0))


def _y1_body(x_ref, w1_ref, dp_ref, o_ref):
    d = _d_of(dp_ref[...])
    o_ref[...] = d * jnp.dot(x_ref[...], w1_ref[...],
                             preferred_element_type=_f32)


def _mid_body(dp_ref, y1_ref, p_ref, b1_ref, w2_ref, o_ref):
    d = _d_of(dp_ref[...])
    p = p_ref[...]
    h = jnp.maximum(d * (p[0] + p[1] + y1_ref[...]) + b1_ref[...], 0.0)
    o_ref[...] = d * jnp.dot(h, w2_ref[...], preferred_element_type=_f32)


def _out_body(dp_ref, y2_ref, p_ref, b2_ref, o_ref):
    d = _d_of(dp_ref[...])
    p = p_ref[...]
    o_ref[...] = d * (p[0] + p[1] + y2_ref[...]) + b2_ref[...]


_y1_call = pl.pallas_call(
    _y1_body, out_shape=jax.ShapeDtypeStruct((NPAD, H), _f32))
_mid_call = pl.pallas_call(
    _mid_body, out_shape=jax.ShapeDtypeStruct((NPAD, H), _f32))
_out_call = pl.pallas_call(
    _out_body, out_shape=jax.ShapeDtypeStruct((NPAD, H), _f32))


def kernel(x, edge_index, W1, b1, W2, b2):
    src = edge_index[0]
    dst = edge_index[1]
    x_pad = jnp.pad(x, ((0, NPAD - N), (0, 0)))

    pad = E2 - E
    ar = jnp.arange(pad, dtype=jnp.int32)
    src_p = jnp.concatenate([src, (ar * 37) % N])
    dst_p = jnp.concatenate([dst, N + (ar % (NPAD - N))])
    dst2d = dst_p.reshape(E2 // CH, CH)

    degp = _deg_kernel(dst2d)
    dp = degp.reshape(NC, NPAD, 1)

    y1 = _y1_call(x_pad, W1, dp)
    p1 = _agg_kernel(y1, src_p, dst2d)
    y2 = _mid_call(dp, y1, p1, b1, W2)
    p2 = _agg_kernel(y2, src_p, dst2d)
    out = _out_call(dp, y2, p2, b2)
    return out[:N]


# speedup vs baseline: 31.4095x; 1.0164x over previous
import functools

import jax
import jax.numpy as jnp
from jax import lax
from jax.experimental import pallas as pl
from jax.experimental.pallas import tpu as pltpu
from jax.experimental.pallas import tpu_sc as plsc

N = 10000
E = 320000
D = 128
H = 128

NC = 2
NS = 16
NW = NC * NS
NPAD = 10240
RPT = NPAD // NS
CH = 64
NCH = 160
HCH = NCH // 2
EPW = NCH * CH
E2 = EPW * NW

_f32 = jnp.float32
_mesh = plsc.VectorSubcoreMesh(core_axis_name="c", subcore_axis_name="s")


def _fill_vmem_1d(ref, n, val):
    def body(i, carry):
        ref[pl.ds(i * 16, 16)] = jnp.full((16,), val, _f32)
        return carry
    lax.fori_loop(0, n // 16, body, 0)


def _zero_vmem_2d(ref, rows):
    def body(i, carry):
        for j in range(D // 16):
            ref[i, pl.ds(j * 16, 16)] = jnp.zeros((16,), _f32)
        return carry
    lax.fori_loop(0, rows, body, 0)


def _deg_body(dst2d_hbm, out_hbm, didx_all, ones_v, zb, deg_sh, dsem):
    c = lax.axis_index("c")
    s = lax.axis_index("s")
    w = s * NC + c

    pltpu.sync_copy(dst2d_hbm.at[pl.ds(w * NCH, NCH)], didx_all)
    _fill_vmem_1d(ones_v, CH, 1.0)
    _fill_vmem_1d(zb, RPT, 0.0)
    pltpu.sync_copy(zb, deg_sh.at[pl.ds(s * RPT, RPT)])
    plsc.subcore_barrier()

    def chunk(j, carry):
        pltpu.async_copy(ones_v, deg_sh.at[didx_all.at[j]], dsem, add=True)

        @pl.when(j >= 8)
        def _():
            pltpu.make_async_copy(ones_v, deg_sh.at[didx_all.at[j - 8]],
                                  dsem).wait()
        return carry

    lax.fori_loop(0, NCH, chunk, 0)
    for k in range(8):
        pltpu.make_async_copy(ones_v, deg_sh.at[didx_all.at[NCH - 8 + k]],
                              dsem).wait()

    plsc.subcore_barrier()
    pltpu.sync_copy(deg_sh.at[pl.ds(s * RPT, RPT)],
                    out_hbm.at[c, pl.ds(s * RPT, RPT)])


_deg_kernel = functools.partial(
    pl.kernel,
    out_type=jax.ShapeDtypeStruct((NC, NPAD), _f32),
    mesh=_mesh,
    scratch_types=[
        pltpu.VMEM((NCH, CH), jnp.int32),
        pltpu.VMEM((CH,), _f32),
        pltpu.VMEM((RPT,), _f32),
        pltpu.VMEM_SHARED((NPAD,), _f32),
        pltpu.SemaphoreType.DMA,
    ],
)(_deg_body)


def _agg_body(y_hbm, src_hbm, dst2d_hbm, out_hbm,
              sidx_h, didx_all, rows0, rows1, rows2, rows3, acc_sh,
              gsem0, gsem1, gsem2, gsem3, ssem0, ssem1, ssem2, ssem3):
    c = lax.axis_index("c")
    s = lax.axis_index("s")
    w = s * NC + c

    rows = (rows0, rows1, rows2, rows3)
    gsem = (gsem0, gsem1, gsem2, gsem3)
    ssem = (ssem0, ssem1, ssem2, ssem3)

    _zero_vmem_2d(rows0, 64)

    def zcp(t, carry):
        pltpu.sync_copy(rows0, acc_sh.at[pl.ds(s * RPT + t * 64, 64)])
        return carry

    lax.fori_loop(0, RPT // 64, zcp, 0)
    plsc.subcore_barrier()

    def gsl(l):
        return sidx_h.at[pl.ds(l * CH, CH)]

    for h in range(2):
        jbase = h * HCH
        pltpu.sync_copy(src_hbm.at[pl.ds(w * EPW + jbase * CH, HCH * CH)],
                        sidx_h)
        pltpu.sync_copy(dst2d_hbm.at[pl.ds(w * NCH + jbase, HCH)], didx_all)
        pltpu.async_copy(y_hbm.at[gsl(0)], rows0, gsem0)
        pltpu.async_copy(y_hbm.at[gsl(1)], rows1, gsem1)

        def quad(t, carry):
            for u in range(4):
                l = 4 * t + u
                b, b2 = u, (u + 2) % 4
                pltpu.make_async_copy(y_hbm.at[gsl(l)], rows[b],
                                      gsem[b]).wait()
                pltpu.async_copy(rows[b], acc_sh.at[didx_all.at[l]],
                                 ssem[b], add=True)

                @pl.when(l >= 2)
                def _():
                    pltpu.make_async_copy(rows[b2],
                                          acc_sh.at[didx_all.at[l - 2]],
                                          ssem[b2]).wait()

                @pl.when(l + 2 < HCH)
                def _():
                    pltpu.async_copy(y_hbm.at[gsl(l + 2)], rows[b2], gsem[b2])
            return carry

        lax.fori_loop(0, HCH // 4, quad, 0)
        pltpu.make_async_copy(rows2, acc_sh.at[didx_all.at[HCH - 2]],
                              ssem2).wait()
        pltpu.make_async_copy(rows3, acc_sh.at[didx_all.at[HCH - 1]],
                              ssem3).wait()

    plsc.subcore_barrier()
    pltpu.sync_copy(acc_sh.at[pl.ds(s * RPT, RPT)],
                    out_hbm.at[c, pl.ds(s * RPT, RPT)])


_agg_kernel = functools.partial(
    pl.kernel,
    out_type=jax.ShapeDtypeStruct((NC, NPAD, D), _f32),
    mesh=_mesh,
    scratch_types=[
        pltpu.VMEM((HCH * CH,), jnp.int32),
        pltpu.VMEM((HCH, CH), jnp.int32),
        pltpu.VMEM((CH, D), _f32),
        pltpu.VMEM((CH, D), _f32),
        pltpu.VMEM((CH, D), _f32),
        pltpu.VMEM((CH, D), _f32),
        pltpu.VMEM_SHARED((NPAD, D), _f32),
        pltpu.SemaphoreType.DMA,
        pltpu.SemaphoreType.DMA,
        pltpu.SemaphoreType.DMA,
        pltpu.SemaphoreType.DMA,
        pltpu.SemaphoreType.DMA,
        pltpu.SemaphoreType.DMA,
        pltpu.SemaphoreType.DMA,
        pltpu.SemaphoreType.DMA,
    ],
)(_agg_body)


def _d_of(dp):
    return lax.rsqrt(jnp.maximum(dp[0] + dp[1] + 1.0, 1.0))


def _y1_body(x_ref, w1_ref, dp_ref, o_ref):
    d = _d_of(dp_ref[...])
    o_ref[pl.ds(0, N)] = d[:N] * jnp.dot(x_ref[...], w1_ref[...],
                                         preferred_element_type=_f32)
    o_ref[pl.ds(N, NPAD - N)] = jnp.zeros((NPAD - N, H), _f32)


def _mid_body(dp_ref, y1_ref, p_ref, b1_ref, w2_ref, o_ref):
    d = _d_of(dp_ref[...])
    p = p_ref[...]
    h = jnp.maximum(d * (p[0] + p[1] + y1_ref[...]) + b1_ref[...], 0.0)
    o_ref[...] = d * jnp.dot(h, w2_ref[...], preferred_element_type=_f32)


def _out_body(dp_ref, y2_ref, p_ref, b2_ref, o_ref):
    d = _d_of(dp_ref[...])[:N]
    p = p_ref[...]
    o_ref[...] = d * (p[0, :N] + p[1, :N] + y2_ref[pl.ds(0, N)]) + b2_ref[...]


_y1_call = pl.pallas_call(
    _y1_body, out_shape=jax.ShapeDtypeStruct((NPAD, H), _f32))
_mid_call = pl.pallas_call(
    _mid_body, out_shape=jax.ShapeDtypeStruct((NPAD, H), _f32))
_out_call = pl.pallas_call(
    _out_body, out_shape=jax.ShapeDtypeStruct((N, H), _f32))


def kernel(x, edge_index, W1, b1, W2, b2):
    src = edge_index[0]
    dst = edge_index[1]

    pad = E2 - E
    ar = jnp.arange(pad, dtype=jnp.int32)
    src_p = jnp.concatenate([src, (ar * 37) % N])
    dst_p = jnp.concatenate([dst, N + (ar % (NPAD - N))])
    dst2d = dst_p.reshape(E2 // CH, CH)

    degp = _deg_kernel(dst2d)
    dp = degp.reshape(NC, NPAD, 1)

    y1 = _y1_call(x, W1, dp)
    p1 = _agg_kernel(y1, src_p, dst2d)
    y2 = _mid_call(dp, y1, p1, b1, W2)
    p2 = _agg_kernel(y2, src_p, dst2d)
    return _out_call(dp, y2, p2, b2)
